# baseline (speedup 1.0000x reference)
import jax
import jax.numpy as jnp
from jax import lax
from jax.experimental import pallas as pl
from jax.experimental.pallas import tpu as pltpu
from jax.experimental.pallas import tpu_sc as plsc

N = 10000
E = 160000
C = 16
K = 5

NP = 10240
EP = 163840
_NROW = EP * C // 128
_NNODE = NP * C // 128

_NC = 2
_NS = 16
_NW = _NC * _NS
_RPW = _NROW // _NW
_CHR = 128
_NCH = _RPW // _CHR
_CHE = _CHR * 128
_EPW = _RPW * 128
_SEG = NP * C // _NS

_mesh = lambda: plsc.VectorSubcoreMesh(core_axis_name="c", subcore_axis_name="s")


def _stream_gather(table_hbm, idx_v, vals_v, sem):
    def body(i, _):
        cps = []
        for b in range(4):
            j = i * 4 + b
            cps.append(pltpu.async_copy(
                table_hbm.at[idx_v.at[j]],
                vals_v.at[pl.ds(j * 128, 128)], sem))
        for cp in cps:
            cp.wait()
        return 0
    lax.fori_loop(0, _CHR // 4, body, 0)


def _gather_body(table_hbm, idxe_hbm, out_hbm, idx_v, vals_v, sem):
    c = lax.axis_index("c")
    s = lax.axis_index("s")
    wid = c * _NS + s
    def chunk(k, _):
        pltpu.sync_copy(idxe_hbm.at[pl.ds(wid * _RPW + k * _CHR, _CHR)],
                        idx_v)
        _stream_gather(table_hbm, idx_v, vals_v, sem)
        pltpu.sync_copy(vals_v,
                        out_hbm.at[pl.ds(wid * _EPW + k * _CHE, _CHE)])
        return 0
    lax.fori_loop(0, _NCH, chunk, 0)


def _sc_gather(table, idxe):
    return pl.kernel(
        _gather_body,
        out_type=jax.ShapeDtypeStruct((EP * C,), jnp.float32),
        mesh=_mesh(),
        scratch_types=[
            pltpu.VMEM((_CHR, 128), jnp.int32),
            pltpu.VMEM((_CHE,), jnp.float32),
            pltpu.SemaphoreType.DMA,
        ],
    )(table, idxe)


def _scatter_phase(msg_hbm, dste_hbm, acc_sh, idx_v, vals_v, wid):
    def chunk(k, _):
        pltpu.sync_copy(msg_hbm.at[pl.ds(wid * _EPW + k * _CHE, _CHE)],
                        vals_v)
        pltpu.sync_copy(dste_hbm.at[pl.ds(wid * _RPW + k * _CHR, _CHR)],
                        idx_v)

        def row(j, _):
            pltpu.sync_copy(vals_v.at[pl.ds(j * 128, 128)],
                            acc_sh.at[idx_v.at[j]], add=True)
            return 0
        lax.fori_loop(0, _CHR, row, 0)
        return 0
    lax.fori_loop(0, _NCH, chunk, 0)


def _scatter_rv_body(msg_hbm, dste_hbm, rve_hbm, zero_hbm,
                     parts_hbm, prev_hbm, acc_sh, idx_v, vals_v, sem):
    c = lax.axis_index("c")
    s = lax.axis_index("s")
    wid = c * _NS + s
    pltpu.sync_copy(zero_hbm.at[pl.ds(s * _SEG, _SEG)],
                    acc_sh.at[pl.ds(s * _SEG, _SEG)])
    plsc.subcore_barrier()
    _scatter_phase(msg_hbm, dste_hbm, acc_sh, idx_v, vals_v, wid)
    def chunk(k, _):
        pltpu.sync_copy(rve_hbm.at[pl.ds(wid * _RPW + k * _CHR, _CHR)],
                        idx_v)
        _stream_gather(msg_hbm, idx_v, vals_v, sem)
        pltpu.sync_copy(vals_v,
                        prev_hbm.at[pl.ds(wid * _EPW + k * _CHE, _CHE)])
        return 0
    lax.fori_loop(0, _NCH, chunk, 0)
    plsc.subcore_barrier()
    pltpu.sync_copy(acc_sh.at[pl.ds(s * _SEG, _SEG)],
                    parts_hbm.at[pl.ds(c * NP * C + s * _SEG, _SEG)])


def _sc_scatter_rv(msg, dste, rve, zero):
    return pl.kernel(
        _scatter_rv_body,
        out_type=[
            jax.ShapeDtypeStruct((_NC * NP * C,), jnp.float32),
            jax.ShapeDtypeStruct((EP * C,), jnp.float32),
        ],
        mesh=_mesh(),
        scratch_types=[
            pltpu.VMEM_SHARED((NP * C,), jnp.float32),
            pltpu.VMEM((_CHR, 128), jnp.int32),
            pltpu.VMEM((_CHE,), jnp.float32),
            pltpu.SemaphoreType.DMA,
        ],
    )(msg, dste, rve, zero)


def _scatter_body(msg_hbm, dste_hbm, zero_hbm, parts_hbm, acc_sh, idx_v,
                  vals_v):
    c = lax.axis_index("c")
    s = lax.axis_index("s")
    wid = c * _NS + s
    pltpu.sync_copy(zero_hbm.at[pl.ds(s * _SEG, _SEG)],
                    acc_sh.at[pl.ds(s * _SEG, _SEG)])
    plsc.subcore_barrier()
    _scatter_phase(msg_hbm, dste_hbm, acc_sh, idx_v, vals_v, wid)
    plsc.subcore_barrier()
    pltpu.sync_copy(acc_sh.at[pl.ds(s * _SEG, _SEG)],
                    parts_hbm.at[pl.ds(c * NP * C + s * _SEG, _SEG)])


def _sc_scatter(msg, dste, zero):
    return pl.kernel(
        _scatter_body,
        out_type=jax.ShapeDtypeStruct((_NC * NP * C,), jnp.float32),
        mesh=_mesh(),
        scratch_types=[
            pltpu.VMEM_SHARED((NP * C,), jnp.float32),
            pltpu.VMEM((_CHR, 128), jnp.int32),
            pltpu.VMEM((_CHE,), jnp.float32),
        ],
    )(msg, dste, zero)


_BR = 4096
_NBK = _NROW // _BR


def _eye16():
    r = lax.broadcasted_iota(jnp.int32, (C, C), 0)
    q = lax.broadcasted_iota(jnp.int32, (C, C), 1)
    return (r == q).astype(jnp.float32)


def _sel(j):
    l = lax.broadcasted_iota(jnp.int32, (128, C), 0)
    c = lax.broadcasted_iota(jnp.int32, (128, C), 1)
    return (l == c + 16 * j).astype(jnp.float32)


def _msg_compute(x, wv, lh, out_ref):
    eye = _eye16()
    acc = jnp.zeros(x.shape, jnp.float32)
    for j in range(8):
        ej = _sel(j)
        zj = lax.dot_general(x, ej, (((1,), (0,)), ((), ())),
                             preferred_element_type=jnp.float32,
                             precision=lax.Precision.HIGHEST)
        zt = lax.dot_general(eye, zj, (((1,), (1,)), ((), ())),
                             preferred_element_type=jnp.float32,
                             precision=lax.Precision.HIGHEST)
        wj = lax.dot_general(wv, ej, (((1,), (0,)), ((), ())),
                             preferred_element_type=jnp.float32,
                             precision=lax.Precision.HIGHEST)
        wt = lax.dot_general(eye, wj, (((1,), (1,)), ((), ())),
                             preferred_element_type=jnp.float32,
                             precision=lax.Precision.HIGHEST)[0:1]
        m = jnp.max(zt, axis=0, keepdims=True)
        p = jnp.exp(zt - m)
        sigs = []
        for c2 in range(C):
            q = jnp.exp(lh[:, c2][:, None] * wt)
            sigs.append(jnp.sum(p * q, axis=0, keepdims=True))
        sig = jnp.concatenate(sigs, axis=0)
        den = jnp.sum(sig, axis=0, keepdims=True)
        lm = jnp.log(sig) - jnp.log(den)
        yj = lax.dot_general(lm, eye, (((0,), (0,)), ((), ())),
                             preferred_element_type=jnp.float32,
                             precision=lax.Precision.HIGHEST)
        acc = acc + lax.dot_general(yj, ej, (((1,), (1,)), ((), ())),
                                    preferred_element_type=jnp.float32,
                             precision=lax.Precision.HIGHEST)
    out_ref[...] = acc


def _msg_body(xj_ref, w_ref, lh_ref, out_ref):
    _msg_compute(xj_ref[...], w_ref[...], lh_ref[...], out_ref)


def _msg_body_sub(xj_ref, prev_ref, w_ref, lh_ref, out_ref):
    _msg_compute(xj_ref[...] - prev_ref[...], w_ref[...], lh_ref[...],
                 out_ref)


def _messages(xj, prev, wfull, logH):
    bspec = pl.BlockSpec((_BR, 128), lambda i: (i, 0))
    hspec = pl.BlockSpec((C, C), lambda i: (0, 0))
    out_sh = jax.ShapeDtypeStruct((_NROW, 128), jnp.float32)
    x2 = xj.reshape(_NROW, 128)
    if prev is None:
        out = pl.pallas_call(
            _msg_body, grid=(_NBK,),
            in_specs=[bspec, bspec, hspec],
            out_specs=bspec, out_shape=out_sh,
        )(x2, wfull, logH)
    else:
        out = pl.pallas_call(
            _msg_body_sub, grid=(_NBK,),
            in_specs=[bspec, bspec, bspec, hspec],
            out_specs=bspec, out_shape=out_sh,
        )(x2, prev.reshape(_NROW, 128), wfull, logH)
    return out.reshape(-1)


_BNX = 1280


def _mlp_body(x_ref, w1_ref, b1_ref, w2_ref, b2_ref, p_ref, h_ref, lh_ref):
    a = jnp.dot(x_ref[...], w1_ref[...], preferred_element_type=jnp.float32,
                             precision=lax.Precision.HIGHEST)
    a = jnp.maximum(a + b1_ref[...], 0.0)
    h = jnp.dot(a, w2_ref[...],
                preferred_element_type=jnp.float32,
                             precision=lax.Precision.HIGHEST) + b2_ref[...]
    rows = _BNX // 8
    acc = jnp.zeros((rows, 128), jnp.float32)
    rr = lax.broadcasted_iota(jnp.int32, (rows, _BNX), 0)
    nn = lax.broadcasted_iota(jnp.int32, (rows, _BNX), 1)
    for j in range(8):
        pj = (nn == 8 * rr + j).astype(jnp.float32)
        hj = lax.dot_general(pj, h, (((1,), (0,)), ((), ())),
                             preferred_element_type=jnp.float32,
                             precision=lax.Precision.HIGHEST)
        acc = acc + lax.dot_general(hj, _sel(j), (((1,), (1,)), ((), ())),
                                    preferred_element_type=jnp.float32,
                             precision=lax.Precision.HIGHEST)
    h_ref[...] = acc
    z = p_ref[...] + p_ref[...].T
    lh_ref[...] = -jnp.log1p(jnp.exp(-jnp.abs(z))) + jnp.minimum(z, 0.0)


def _mlp(xpad, W1, b1, W2, b2, param):
    nb = NP // _BNX
    return pl.pallas_call(
        _mlp_body, grid=(nb,),
        in_specs=[
            pl.BlockSpec((_BNX, 128), lambda i: (i, 0)),
            pl.BlockSpec((128, 32), lambda i: (0, 0)),
            pl.BlockSpec((1, 32), lambda i: (0, 0)),
            pl.BlockSpec((32, C), lambda i: (0, 0)),
            pl.BlockSpec((1, C), lambda i: (0, 0)),
            pl.BlockSpec((C, C), lambda i: (0, 0)),
        ],
        out_specs=[
            pl.BlockSpec((_BNX // 8, 128), lambda i: (i, 0)),
            pl.BlockSpec((C, C), lambda i: (0, 0)),
        ],
        out_shape=[
            jax.ShapeDtypeStruct((_NNODE, 128), jnp.float32),
            jax.ShapeDtypeStruct((C, C), jnp.float32),
        ],
    )(xpad, W1, b1.reshape(1, 32), W2, b2.reshape(1, C), param)


_BNN = 128


def _combine_body(h_ref, p0_ref, p1_ref, out_ref):
    out_ref[...] = h_ref[...] + p0_ref[0] + p1_ref[0]


def _finalize_body(h_ref, p0_ref, p1_ref, out_ref):
    raw = h_ref[...] + p0_ref[0] + p1_ref[0]
    eye = _eye16()
    acc = jnp.zeros(raw.shape, jnp.float32)
    for j in range(8):
        ej = _sel(j)
        zj = lax.dot_general(raw, ej, (((1,), (0,)), ((), ())),
                             preferred_element_type=jnp.float32,
                             precision=lax.Precision.HIGHEST)
        zt = lax.dot_general(eye, zj, (((1,), (1,)), ((), ())),
                             preferred_element_type=jnp.float32,
                             precision=lax.Precision.HIGHEST)
        m = jnp.max(zt, axis=0, keepdims=True)
        lse = m + jnp.log(jnp.sum(jnp.exp(zt - m), axis=0, keepdims=True))
        yj = lax.dot_general(zt - lse, eye, (((0,), (0,)), ((), ())),
                             preferred_element_type=jnp.float32,
                             precision=lax.Precision.HIGHEST)
        acc = acc + lax.dot_general(yj, ej, (((1,), (1,)), ((), ())),
                                    preferred_element_type=jnp.float32,
                             precision=lax.Precision.HIGHEST)
    out_ref[...] = acc


def _node_combine(body, hf, parts):
    nb = _NNODE // _BNN
    p3 = parts.reshape(_NC, _NNODE, 128)
    nspec = pl.BlockSpec((_BNN, 128), lambda i: (i, 0))
    return pl.pallas_call(
        body, grid=(nb,),
        in_specs=[
            nspec,
            pl.BlockSpec((1, _BNN, 128), lambda i: (0, i, 0)),
            pl.BlockSpec((1, _BNN, 128), lambda i: (1, i, 0)),
        ],
        out_specs=nspec,
        out_shape=jax.ShapeDtypeStruct((_NNODE, 128), jnp.float32),
    )(hf, p3, p3)


def kernel(x, edge_index, edge_weight, edge_rv, W1, b1, W2, b2, param):
    xpad = jnp.pad(x, ((0, NP - N), (0, 0)))
    lane = jnp.arange(C, dtype=jnp.int32)[None, :]
    src_p = jnp.pad(edge_index[0], (0, EP - E))
    dst_p = jnp.pad(edge_index[1], (0, EP - E), constant_values=N)
    rv_p = jnp.pad(edge_rv, (0, EP - E))
    srce = (src_p[:, None] * C + lane).reshape(_NROW, 128)
    dste = (dst_p[:, None] * C + lane).reshape(_NROW, 128)
    rve = (rv_p[:, None] * C + lane).reshape(_NROW, 128)
    w_p = jnp.pad(edge_weight, (0, EP - E))
    wfull = jnp.broadcast_to(w_p[:, None], (EP, C)).reshape(_NROW, 128)
    zero = jnp.zeros((NP * C,), jnp.float32)

    srcef = srce.reshape(-1)
    rvef = rve.reshape(-1)
    hf, logH = _mlp(xpad, W1, b1, W2, b2, param)
    h_flat = hf.reshape(-1)
    xj = jnp.take(h_flat, srcef, axis=0)
    msg = _messages(xj, None, wfull, logH)
    dstef = dste.reshape(-1)
    for _ in range(K - 1):
        agg = jax.ops.segment_sum(msg, dstef, num_segments=NP * C)
        parts = jnp.concatenate([agg, jnp.zeros_like(agg)])
        prev = jnp.take(msg, rvef, axis=0)
        raw = _node_combine(_combine_body, hf, parts.reshape(-1, 128))
        xj = jnp.take(raw.reshape(-1), srcef, axis=0)
        msg = _messages(xj, prev, wfull, logH)
    agg = jax.ops.segment_sum(msg, dstef, num_segments=NP * C)
    parts = jnp.concatenate([agg, jnp.zeros_like(agg)])
    out_flat = _node_combine(_finalize_body, hf, parts.reshape(-1, 128))
    return out_flat.reshape(NP, C)[:N]
